# baseline (speedup 1.0000x reference)
import functools

import jax
import jax.numpy as jnp
from jax import lax
from jax.experimental import pallas as pl
from jax.experimental.pallas import tpu as pltpu
from jax.experimental.pallas import tpu_sc as plsc

V = 8192
NX = 2048
NY = 2048
L = 16
NC = 2
NS = 16
NW = NC * NS
ROWS_PER_W = NX // NW
B = 4
NBUF = 3
NBATCH = ROWS_PER_W // B


def _sc_body(dist_hbm, x_hbm, y_hbm, nis_hbm, out_hbm,
             xv, yv, nisv, rows, outb, isems, osems):
    wid = lax.axis_index("s") * NC + lax.axis_index("c")
    base = wid * ROWS_PER_W

    pltpu.sync_copy(x_hbm.at[wid], xv)
    pltpu.sync_copy(y_hbm, yv)
    pltpu.sync_copy(nis_hbm, nisv)
    nis = nisv[...]

    rsels = [jnp.full((L,), r, jnp.int32) for r in range(B)]

    def fire_in(k):
        buf = k % NBUF
        return pltpu.async_copy(
            dist_hbm.at[xv.at[k]], rows.at[buf], isems[buf])

    in_descs = [None] * NBUF
    out_descs = [None] * NBUF
    in_descs[0] = fire_in(0)

    for k in range(NBATCH):
        buf = k % NBUF
        in_descs[buf].wait()
        if k + 1 < NBATCH:
            in_descs[(k + 1) % NBUF] = fire_in(k + 1)
        if out_descs[buf] is not None:
            out_descs[buf].wait()

        @plsc.parallel_loop(0, NY, step=L, unroll=8)
        def col_body(jj):
            idx = yv[pl.ds(jj, L)]
            for r in range(B):
                g = plsc.load_gather(rows.at[buf], [rsels[r], idx])
                outb[buf, r, pl.ds(jj, L)] = jnp.exp(g * nis)
        out_descs[buf] = pltpu.async_copy(
            outb.at[buf], out_hbm.at[pl.ds(base + k * B, B)], osems[buf])

    for d in out_descs:
        d.wait()


_sc_call = functools.partial(
    pl.kernel,
    out_type=jax.ShapeDtypeStruct((NX, NY), jnp.float32),
    mesh=plsc.VectorSubcoreMesh(
        core_axis_name="c", subcore_axis_name="s",
        num_cores=NC, num_subcores=NS),
    scratch_types=[
        pltpu.VMEM((NBATCH, B), jnp.int32),
        pltpu.VMEM((NY,), jnp.int32),
        pltpu.VMEM((L,), jnp.float32),
        pltpu.VMEM((NBUF, B, V), jnp.float32),
        pltpu.VMEM((NBUF, B, NY), jnp.float32),
        [pltpu.SemaphoreType.DMA] * NBUF,
        [pltpu.SemaphoreType.DMA] * NBUF,
    ],
    compiler_params=pltpu.CompilerParams(
        use_tc_tiling_on_sc=True, needs_layout_passes=False),
)(_sc_body)


def kernel(x, y, distance, scale):
    s = jnp.clip(jax.nn.softplus(scale), 1e-10, 10000.0)
    nis = jnp.broadcast_to((-1.0 / s).astype(jnp.float32), (L,))
    xr = x.astype(jnp.int32).reshape(NW, NBATCH, B)
    return _sc_call(distance, xr, y.astype(jnp.int32), nis)


# speedup vs baseline: 1.0751x; 1.0751x over previous
import functools

import jax
import jax.numpy as jnp
from jax import lax
from jax.experimental import pallas as pl
from jax.experimental.pallas import tpu as pltpu
from jax.experimental.pallas import tpu_sc as plsc

V = 8192
NX = 2048
NY = 2048
L = 16
NC = 2
NS = 16
NW = NC * NS
ROWS_PER_W = NX // NW
B = 4
NBUF = 3
NBATCH = ROWS_PER_W // B


def _sc_body(dist_hbm, x_hbm, y_hbm, nis_hbm, out_hbm,
             xv, yv, nisv, rows, outb, isems, osems):
    wid = lax.axis_index("s") * NC + lax.axis_index("c")
    base = wid * ROWS_PER_W

    pltpu.sync_copy(x_hbm.at[wid], xv)
    pltpu.sync_copy(y_hbm, yv)
    pltpu.sync_copy(nis_hbm, nisv)
    nis = nisv[...]

    rsels = [jnp.full((L,), r, jnp.int32) for r in range(B)]

    def fire_in(k):
        buf = k % NBUF
        return pltpu.async_copy(
            dist_hbm.at[xv.at[k]], rows.at[buf], isems[buf])

    in_descs = [None] * NBUF
    out_descs = [None] * NBUF
    for k in range(NBUF - 1):
        in_descs[k] = fire_in(k)

    for k in range(NBATCH):
        buf = k % NBUF
        if k + NBUF - 1 < NBATCH:
            in_descs[(k + NBUF - 1) % NBUF] = fire_in(k + NBUF - 1)
        in_descs[buf].wait()
        if out_descs[buf] is not None:
            out_descs[buf].wait()

        @plsc.parallel_loop(0, NY, step=L, unroll=8)
        def col_body(jj):
            idx = yv[pl.ds(jj, L)]
            for r in range(B):
                g = plsc.load_gather(rows.at[buf], [rsels[r], idx])
                outb[buf, r, pl.ds(jj, L)] = jnp.exp(g * nis)
        out_descs[buf] = pltpu.async_copy(
            outb.at[buf], out_hbm.at[pl.ds(base + k * B, B)], osems[buf])

    for d in out_descs:
        d.wait()


_sc_call = functools.partial(
    pl.kernel,
    out_type=jax.ShapeDtypeStruct((NX, NY), jnp.float32),
    mesh=plsc.VectorSubcoreMesh(
        core_axis_name="c", subcore_axis_name="s",
        num_cores=NC, num_subcores=NS),
    scratch_types=[
        pltpu.VMEM((NBATCH, B), jnp.int32),
        pltpu.VMEM((NY,), jnp.int32),
        pltpu.VMEM((L,), jnp.float32),
        pltpu.VMEM((NBUF, B, V), jnp.float32),
        pltpu.VMEM((NBUF, B, NY), jnp.float32),
        [pltpu.SemaphoreType.DMA] * NBUF,
        [pltpu.SemaphoreType.DMA] * NBUF,
    ],
    compiler_params=pltpu.CompilerParams(
        use_tc_tiling_on_sc=True, needs_layout_passes=False),
)(_sc_body)


def kernel(x, y, distance, scale):
    s = jnp.clip(jax.nn.softplus(scale), 1e-10, 10000.0)
    nis = jnp.broadcast_to((-1.0 / s).astype(jnp.float32), (L,))
    xr = x.astype(jnp.int32).reshape(NW, NBATCH, B)
    return _sc_call(distance, xr, y.astype(jnp.int32), nis)
